# baseline (speedup 1.0000x reference)
import functools

import numpy as np

import jax
import jax.numpy as jnp
from jax import lax
from jax.experimental import pallas as pl
from jax.experimental.pallas import tpu as pltpu
from jax.experimental.pallas import tpu_sc as plsc

N = 10000
D = 128
DH = D // 2
E = 320000
NC, NS = 2, 16
LANE = 128
CHUNKS = 160
QRT = CHUNKS // 4
EPAD = NS * CHUNKS * LANE
NPAD = 10240
RPS = NPAD // NS
BR = 1024

_ZEROS_N = np.zeros((NPAD,), np.float32)
_ZEROS_NH = np.zeros((NPAD, DH), np.float32)
_ONES_L = np.ones((LANE,), np.float32)
_FILL = np.full((EPAD - E,), N, np.int32)


@functools.lru_cache(maxsize=None)
def _sc_kernels():
    mesh = plsc.VectorSubcoreMesh(
        core_axis_name="c", subcore_axis_name="s", num_cores=NC, num_subcores=NS)

    @functools.partial(
        pl.kernel,
        out_type=jax.ShapeDtypeStruct((NC, NPAD), jnp.float32),
        mesh=mesh,
        scratch_types=[
            pltpu.VMEM((CHUNKS // NC, LANE), jnp.int32),
            pltpu.VMEM((LANE,), jnp.float32),
            pltpu.VMEM_SHARED((NPAD,), jnp.float32),
        ],
    )
    def deg_kernel(dst_hbm, zeros_hbm, ones_hbm, hist_out, idx_v, ones_v, hist_sh):
        c = lax.axis_index("c")
        s = lax.axis_index("s")
        half = CHUNKS // NC
        pltpu.sync_copy(zeros_hbm.at[pl.ds(s * RPS, RPS)],
                        hist_sh.at[pl.ds(s * RPS, RPS)])
        pltpu.sync_copy(ones_hbm, ones_v)
        pltpu.sync_copy(dst_hbm.at[s, pl.ds(c * half, half)], idx_v)
        plsc.subcore_barrier()

        def body(j, carry):
            pltpu.sync_copy(ones_v, hist_sh.at[idx_v.at[j]], add=True)
            return carry

        lax.fori_loop(0, half, body, 0)
        plsc.subcore_barrier()

        @pl.when(s == 0)
        def _():
            pltpu.sync_copy(hist_sh, hist_out.at[c])

    @functools.partial(
        pl.kernel,
        out_type=jax.ShapeDtypeStruct((NC, NPAD, DH), jnp.float32),
        mesh=mesh,
        compiler_params=pltpu.CompilerParams(use_tc_tiling_on_sc=False),
        scratch_types=[
            pltpu.VMEM((QRT, LANE), jnp.int32),
            pltpu.VMEM((QRT, LANE), jnp.int32),
            pltpu.VMEM((LANE, DH), jnp.float32),
            pltpu.VMEM((LANE, DH), jnp.float32),
            pltpu.VMEM_SHARED((NPAD, DH), jnp.float32),
            pltpu.VMEM_SHARED((NPAD, DH), jnp.float32),
            pltpu.SemaphoreType.DMA,
            pltpu.SemaphoreType.DMA,
        ],
    )
    def scatter_kernel(g2_hbm, src_hbm, dst_hbm, zeros_hbm, part_out,
                       src_v, dst_v, buf0, buf1, g_sh, acc_sh, sem0, sem1):
        c = lax.axis_index("c")
        s = lax.axis_index("s")
        rows = pl.ds(s * RPS, RPS)

        pltpu.sync_copy(g2_hbm.at[c].at[rows], g_sh.at[rows])
        pltpu.sync_copy(zeros_hbm.at[rows], acc_sh.at[rows])
        plsc.subcore_barrier()

        bufs = (buf0, buf1)
        sems = (sem0, sem1)
        for q in range(CHUNKS // QRT):
            pltpu.sync_copy(src_hbm.at[s, pl.ds(q * QRT, QRT)], src_v)
            pltpu.sync_copy(dst_hbm.at[s, pl.ds(q * QRT, QRT)], dst_v)
            pltpu.make_async_copy(g_sh.at[src_v.at[0]], bufs[0], sems[0]).start()

            def body(p, carry):
                for k in range(2):
                    j = p * 2 + k

                    @pl.when(j + 1 < QRT)
                    def _():
                        pltpu.make_async_copy(
                            g_sh.at[src_v.at[j + 1]], bufs[1 - k], sems[1 - k]).start()

                    pltpu.make_async_copy(
                        g_sh.at[src_v.at[j]], bufs[k], sems[k]).wait()
                    pltpu.sync_copy(bufs[k], acc_sh.at[dst_v.at[j]], add=True)
                return carry

            lax.fori_loop(0, QRT // 2, body, 0)

        plsc.subcore_barrier()
        pltpu.sync_copy(acc_sh.at[rows], part_out.at[c].at[rows])

    return deg_kernel, scatter_kernel


def _mm_body(x_ref, w_ref, hp_ref, g_ref):
    deg = hp_ref[:, 0:1] + hp_ref[:, 1:2] + 1.0
    dinv = lax.rsqrt(deg)
    h = jnp.dot(x_ref[...], w_ref[...], preferred_element_type=jnp.float32)
    t = h * dinv
    g_ref[0] = t[:, :DH]
    g_ref[1] = t[:, DH:]


BRM = 400
_mm = pl.pallas_call(
    _mm_body,
    grid=(N // BRM,),
    in_specs=[
        pl.BlockSpec((BRM, D), lambda i: (i, 0)),
        pl.BlockSpec((D, D), lambda i: (0, 0)),
        pl.BlockSpec((BRM, 2), lambda i: (i, 0)),
    ],
    out_specs=pl.BlockSpec((2, BRM, DH), lambda i: (0, i, 0)),
    out_shape=jax.ShapeDtypeStruct((2, NPAD, DH), jnp.float32),
)


def _fin_body(p_ref, g_ref, hp_ref, b_ref, o_ref):
    deg = hp_ref[:, 0:1] + hp_ref[:, 1:2] + 1.0
    dinv = lax.rsqrt(deg)
    t = p_ref[...] + g_ref[...]
    o_ref[...] = dinv * jnp.concatenate([t[0], t[1]], axis=-1) + b_ref[...]


_fin = pl.pallas_call(
    _fin_body,
    grid=(NPAD // BR,),
    in_specs=[
        pl.BlockSpec((NC, BR, DH), lambda i: (0, i, 0)),
        pl.BlockSpec((2, BR, DH), lambda i: (0, i, 0)),
        pl.BlockSpec((BR, 2), lambda i: (i, 0)),
        pl.BlockSpec((1, D), lambda i: (0, 0)),
    ],
    out_specs=pl.BlockSpec((BR, D), lambda i: (i, 0)),
    out_shape=jax.ShapeDtypeStruct((NPAD, D), jnp.float32),
)


def kernel(mpg_ft, edge_index, W, b):
    src = edge_index[0]
    dst = edge_index[1]
    src_p = jnp.concatenate([src, _FILL]).reshape(NS, CHUNKS, LANE)
    dst_p = jnp.concatenate([dst, _FILL]).reshape(NS, CHUNKS, LANE)

    deg_kernel, scatter_kernel = _sc_kernels()
    hist = deg_kernel(dst_p, _ZEROS_N, _ONES_L)
    hist_pair = jnp.transpose(hist)
    g2 = _mm(mpg_ft, W, hist_pair)
    part = scatter_kernel(g2, src_p, dst_p, _ZEROS_NH)
    out = _fin(part, g2, hist_pair, jnp.reshape(b, (1, D)))
    return out[:N]


# speedup vs baseline: 1.0328x; 1.0328x over previous
import functools

import numpy as np

import jax
import jax.numpy as jnp
from jax import lax
from jax.experimental import pallas as pl
from jax.experimental.pallas import tpu as pltpu
from jax.experimental.pallas import tpu_sc as plsc

N = 10000
D = 128
DH = D // 2
E = 320000
NC, NS = 2, 16
LANE = 128
CHUNKS = 160
QRT = CHUNKS // 4
EPAD = NS * CHUNKS * LANE
NPAD = 10240
RPS = NPAD // NS
BR = 1024

_ZEROS_N = np.zeros((NPAD,), np.float32)
_ZEROS_NH = np.zeros((NPAD, DH), np.float32)
_ONES_L = np.ones((LANE,), np.float32)
_FILL = np.full((EPAD - E,), N, np.int32)


@functools.lru_cache(maxsize=None)
def _sc_kernels():
    mesh = plsc.VectorSubcoreMesh(
        core_axis_name="c", subcore_axis_name="s", num_cores=NC, num_subcores=NS)

    @functools.partial(
        pl.kernel,
        out_type=jax.ShapeDtypeStruct((NC, NPAD), jnp.float32),
        mesh=mesh,
        scratch_types=[
            pltpu.VMEM((CHUNKS // NC, LANE), jnp.int32),
            pltpu.VMEM((LANE,), jnp.float32),
            pltpu.VMEM_SHARED((NPAD,), jnp.float32),
        ],
    )
    def deg_kernel(dst_hbm, zeros_hbm, ones_hbm, hist_out, idx_v, ones_v, hist_sh):
        c = lax.axis_index("c")
        s = lax.axis_index("s")
        half = CHUNKS // NC
        pltpu.sync_copy(zeros_hbm.at[pl.ds(s * RPS, RPS)],
                        hist_sh.at[pl.ds(s * RPS, RPS)])
        pltpu.sync_copy(ones_hbm, ones_v)
        pltpu.sync_copy(dst_hbm.at[s, pl.ds(c * half, half)], idx_v)
        plsc.subcore_barrier()

        def body(j, carry):
            pltpu.sync_copy(ones_v, hist_sh.at[idx_v.at[j]], add=True)
            return carry

        lax.fori_loop(0, half, body, 0)
        plsc.subcore_barrier()

        @pl.when(s == 0)
        def _():
            pltpu.sync_copy(hist_sh, hist_out.at[c])

    @functools.partial(
        pl.kernel,
        out_type=jax.ShapeDtypeStruct((NC, NPAD, DH), jnp.float32),
        mesh=mesh,
        compiler_params=pltpu.CompilerParams(use_tc_tiling_on_sc=False),
        scratch_types=[
            pltpu.VMEM((QRT, LANE), jnp.int32),
            pltpu.VMEM((QRT, LANE), jnp.int32),
            pltpu.VMEM((LANE, DH), jnp.float32),
            pltpu.VMEM((LANE, DH), jnp.float32),
            pltpu.VMEM_SHARED((NPAD, DH), jnp.float32),
            pltpu.VMEM_SHARED((NPAD, DH), jnp.float32),
            pltpu.SemaphoreType.DMA,
            pltpu.SemaphoreType.DMA,
        ],
    )
    def scatter_kernel(g2_hbm, src_hbm, dst_hbm, zeros_hbm, part_out,
                       src_v, dst_v, buf0, buf1, g_sh, acc_sh, sem0, sem1):
        c = lax.axis_index("c")
        s = lax.axis_index("s")
        rows = pl.ds(s * RPS, RPS)

        pltpu.sync_copy(g2_hbm.at[c].at[rows], g_sh.at[rows])
        pltpu.sync_copy(zeros_hbm.at[rows], acc_sh.at[rows])
        plsc.subcore_barrier()

        bufs = (buf0, buf1)
        sems = (sem0, sem1)
        for q in range(CHUNKS // QRT):
            pltpu.sync_copy(src_hbm.at[s, pl.ds(q * QRT, QRT)], src_v)
            pltpu.sync_copy(dst_hbm.at[s, pl.ds(q * QRT, QRT)], dst_v)
            pltpu.make_async_copy(g_sh.at[src_v.at[0]], bufs[0], sems[0]).start()

            def body(p, carry):
                for k in range(2):
                    j = p * 2 + k

                    @pl.when(j + 1 < QRT)
                    def _():
                        pltpu.make_async_copy(
                            g_sh.at[src_v.at[j + 1]], bufs[1 - k], sems[1 - k]).start()

                    pltpu.make_async_copy(
                        g_sh.at[src_v.at[j]], bufs[k], sems[k]).wait()
                    pltpu.sync_copy(bufs[k], acc_sh.at[dst_v.at[j]], add=True)
                return carry

            lax.fori_loop(0, QRT // 2, body, 0)

        plsc.subcore_barrier()
        pltpu.sync_copy(acc_sh.at[rows], part_out.at[c].at[rows])

    return deg_kernel, scatter_kernel


def _mm_body(x_ref, w_ref, hp_ref, g_ref):
    deg = hp_ref[:, 0:1] + hp_ref[:, 1:2] + 1.0
    dinv = lax.rsqrt(deg)
    h = jnp.dot(x_ref[...], w_ref[...], preferred_element_type=jnp.float32)
    t = h * dinv
    g_ref[0] = t[:, :DH]
    g_ref[1] = t[:, DH:]


BRM = 1000
_mm = pl.pallas_call(
    _mm_body,
    grid=(N // BRM,),
    in_specs=[
        pl.BlockSpec((BRM, D), lambda i: (i, 0)),
        pl.BlockSpec((D, D), lambda i: (0, 0)),
        pl.BlockSpec((BRM, 2), lambda i: (i, 0)),
    ],
    out_specs=pl.BlockSpec((2, BRM, DH), lambda i: (0, i, 0)),
    out_shape=jax.ShapeDtypeStruct((2, NPAD, DH), jnp.float32),
)


def _fin_body(p_ref, g_ref, hp_ref, b_ref, o_ref):
    deg = hp_ref[:, 0:1] + hp_ref[:, 1:2] + 1.0
    dinv = lax.rsqrt(deg)
    t = p_ref[...] + g_ref[...]
    o_ref[...] = dinv * jnp.concatenate([t[0], t[1]], axis=-1) + b_ref[...]


_fin = pl.pallas_call(
    _fin_body,
    grid=(NPAD // BR,),
    in_specs=[
        pl.BlockSpec((NC, BR, DH), lambda i: (0, i, 0)),
        pl.BlockSpec((2, BR, DH), lambda i: (0, i, 0)),
        pl.BlockSpec((BR, 2), lambda i: (i, 0)),
        pl.BlockSpec((1, D), lambda i: (0, 0)),
    ],
    out_specs=pl.BlockSpec((BR, D), lambda i: (i, 0)),
    out_shape=jax.ShapeDtypeStruct((NPAD, D), jnp.float32),
)


def kernel(mpg_ft, edge_index, W, b):
    src = edge_index[0]
    dst = edge_index[1]
    src_p = jnp.concatenate([src, _FILL]).reshape(NS, CHUNKS, LANE)
    dst_p = jnp.concatenate([dst, _FILL]).reshape(NS, CHUNKS, LANE)

    deg_kernel, scatter_kernel = _sc_kernels()
    hist = deg_kernel(dst_p, _ZEROS_N, _ONES_L)
    hist_pair = jnp.transpose(hist)
    g2 = _mm(mpg_ft, W, hist_pair)
    part = scatter_kernel(g2, src_p, dst_p, _ZEROS_NH)
    out = _fin(part, g2, hist_pair, jnp.reshape(b, (1, D)))
    return out[:N]
